# baseline (speedup 1.0000x reference)
import functools

import jax
import jax.numpy as jnp
from jax import lax
from jax.experimental import pallas as pl
from jax.experimental.pallas import tpu as pltpu
from jax.experimental.pallas import tpu_sc as plsc

B, M, D, W = 64, 8192, 128, 33
WP = 40
PAD = WP - W
NWORKERS = 32
BPW = B // NWORKERS


def _copy_patch_body(idx_s, src_ref, rows_ref, dst_ref):
    b = pl.program_id(0)
    dst_ref[...] = src_ref[...]
    for k in range(WP):
        r = idx_s[b, k]
        dst_ref[0, pl.ds(r, 1), :] = rows_ref[0, pl.ds(k, 1), :]


_tc_copy_patch = pl.pallas_call(
    _copy_patch_body,
    grid_spec=pltpu.PrefetchScalarGridSpec(
        num_scalar_prefetch=1,
        grid=(B,),
        in_specs=[
            pl.BlockSpec((1, M, D), lambda b, s: (b, 0, 0)),
            pl.BlockSpec((1, WP, D), lambda b, s: (b, 0, 0)),
        ],
        out_specs=pl.BlockSpec((1, M, D), lambda b, s: (b, 0, 0)),
    ),
    out_shape=jax.ShapeDtypeStruct((B, M, D), jnp.float32),
)

_sc_mesh = plsc.VectorSubcoreMesh(core_axis_name="c", subcore_axis_name="s")


@functools.partial(
    pl.kernel,
    mesh=_sc_mesh,
    out_type=jax.ShapeDtypeStruct((B, WP, D), jnp.float32),
    scratch_types=[
        pltpu.VMEM((WP,), jnp.int32),
        pltpu.VMEM((WP, D), jnp.float32),
        pltpu.VMEM((WP, D), jnp.float32),
        pltpu.VMEM((WP, D), jnp.float32),
        pltpu.SemaphoreType.DMA,
    ],
)
def _sc_rows(ring, idxp, s1p, s2p, rows_out, idx_v, rows_v, s1_v, s2_v, sem):
    wid = lax.axis_index("s") * 2 + lax.axis_index("c")
    for j in range(BPW):
        b = wid * BPW + j
        pltpu.sync_copy(idxp.at[b], idx_v)
        pltpu.async_copy(ring.at[b].at[idx_v], rows_v, sem).wait()
        pltpu.sync_copy(s1p.at[b], s1_v)
        pltpu.sync_copy(s2p.at[b], s2_v)
        for r in range(WP):
            for c in range(D // 16):
                sl = (r, pl.ds(c * 16, 16))
                rows_v[sl] = rows_v[sl] * s1_v[sl] + s2_v[sl]
        pltpu.sync_copy(rows_v, rows_out.at[b])


def kernel(ring, write_vec, idx, weights, erase, write_gate):
    idx = idx.astype(jnp.int32)
    idxp = jnp.concatenate([jnp.broadcast_to(idx[:, -1:], (B, PAD)), idx], axis=1)
    wp = jnp.concatenate(
        [jnp.broadcast_to(weights[:, -1:], (B, PAD)), weights], axis=1)
    eq = idxp[:, :, None] == idxp[:, None, :]
    lastk = jnp.max(jnp.where(eq, jnp.arange(WP)[None, None, :], -1), axis=-1)
    s1 = 1.0 - erase[:, None] * wp
    s2 = write_gate[:, None] * wp
    s1d = jnp.take_along_axis(s1, lastk, axis=1)
    s2d = jnp.take_along_axis(s2, lastk, axis=1)
    s1p = jnp.broadcast_to(s1d[:, :, None], (B, WP, D))
    s2p = s2d[:, :, None] * write_vec[:, None, :]
    rows_upd = _sc_rows(ring, idxp, s1p, s2p)
    return _tc_copy_patch(idxp, ring, rows_upd)


# speedup vs baseline: 1.0481x; 1.0481x over previous
import functools

import jax
import jax.numpy as jnp
from jax import lax
from jax.experimental import pallas as pl
from jax.experimental.pallas import tpu as pltpu
from jax.experimental.pallas import tpu_sc as plsc

B, M, D, W = 64, 8192, 128, 33
WP = 40
PAD = WP - W
NWORKERS = 32
BPW = B // NWORKERS


def _copy_body(src_ref, dst_ref):
    dst_ref[...] = src_ref[...]


_tc_copy = pl.pallas_call(
    _copy_body,
    grid=(B,),
    in_specs=[pl.BlockSpec((1, M, D), lambda b: (b, 0, 0))],
    out_specs=pl.BlockSpec((1, M, D), lambda b: (b, 0, 0)),
    out_shape=jax.ShapeDtypeStruct((B, M, D), jnp.float32),
)

_sc_mesh = plsc.VectorSubcoreMesh(core_axis_name="c", subcore_axis_name="s")


@functools.partial(
    pl.kernel,
    mesh=_sc_mesh,
    out_type=jax.ShapeDtypeStruct((B, WP, D), jnp.float32),
    scratch_types=[
        pltpu.VMEM((WP,), jnp.int32),
        pltpu.VMEM((WP, D), jnp.float32),
        pltpu.VMEM((WP, D), jnp.float32),
        pltpu.VMEM((WP, D), jnp.float32),
        pltpu.SemaphoreType.DMA,
    ],
)
def _sc_rows(ring, idxp, s1p, s2p, rows_out, idx_v, rows_v, s1_v, s2_v, sem):
    wid = lax.axis_index("s") * 2 + lax.axis_index("c")
    for j in range(BPW):
        b = wid * BPW + j
        pltpu.sync_copy(idxp.at[b], idx_v)
        pltpu.async_copy(ring.at[b].at[idx_v], rows_v, sem).wait()
        pltpu.sync_copy(s1p.at[b], s1_v)
        pltpu.sync_copy(s2p.at[b], s2_v)
        for r in range(WP):
            for c in range(D // 16):
                sl = (r, pl.ds(c * 16, 16))
                rows_v[sl] = rows_v[sl] * s1_v[sl] + s2_v[sl]
        pltpu.sync_copy(rows_v, rows_out.at[b])


@functools.partial(
    pl.kernel,
    mesh=_sc_mesh,
    out_type=(),
    scratch_types=[
        pltpu.VMEM((WP,), jnp.int32),
        pltpu.VMEM((WP, D), jnp.float32),
        pltpu.SemaphoreType.DMA,
    ],
)
def _sc_scatter(out_hbm, rows, idxp, idx_v, rows_v, sem):
    wid = lax.axis_index("s") * 2 + lax.axis_index("c")
    for j in range(BPW):
        b = wid * BPW + j
        pltpu.sync_copy(idxp.at[b], idx_v)
        pltpu.sync_copy(rows.at[b], rows_v)
        pltpu.async_copy(rows_v, out_hbm.at[b].at[idx_v], sem).wait()


def kernel(ring, write_vec, idx, weights, erase, write_gate):
    idx = idx.astype(jnp.int32)
    idxp = jnp.concatenate([jnp.broadcast_to(idx[:, -1:], (B, PAD)), idx], axis=1)
    wp = jnp.concatenate(
        [jnp.broadcast_to(weights[:, -1:], (B, PAD)), weights], axis=1)
    eq = idxp[:, :, None] == idxp[:, None, :]
    lastk = jnp.max(jnp.where(eq, jnp.arange(WP)[None, None, :], -1), axis=-1)
    s1 = 1.0 - erase[:, None] * wp
    s2 = write_gate[:, None] * wp
    s1d = jnp.take_along_axis(s1, lastk, axis=1)
    s2d = jnp.take_along_axis(s2, lastk, axis=1)
    s1p = jnp.broadcast_to(s1d[:, :, None], (B, WP, D))
    s2p = s2d[:, :, None] * write_vec[:, None, :]
    rows_upd = _sc_rows(ring, idxp, s1p, s2p)
    out = _tc_copy(ring)
    o_ref = jax.new_ref(out)
    _sc_scatter(o_ref, rows_upd, idxp)
    return jax.freeze(o_ref)
